# baseline (speedup 1.0000x reference)
import functools

import jax
import jax.numpy as jnp
from jax.experimental import pallas as pl
from jax.experimental.pallas import tpu as pltpu


def _se_gate_kernel(x_ref, w1t_ref, b1_ref, w2t_ref, b2_ref, o_ref, *, inv_hw):
    x = x_ref[0]

    pooled = jnp.sum(x, axis=1, keepdims=True) * inv_hw

    h = jnp.dot(w1t_ref[...], pooled,
                preferred_element_type=jnp.float32) + b1_ref[...]
    h = jnp.maximum(h, 0.0)
    y = jax.nn.sigmoid(
        jnp.dot(w2t_ref[...], h,
                preferred_element_type=jnp.float32) + b2_ref[...])

    o_ref[0] = x * y


def kernel(x, w1, b1, w2, b2):
    N, C, H, W = x.shape
    Cr = w1.shape[1]
    HW = H * W

    x_flat = x.reshape(N, C, HW)
    w1t = w1.T
    w2t = w2.T
    b1c = b1.reshape(Cr, 1)
    b2c = b2.reshape(C, 1)

    out_flat = pl.pallas_call(
        functools.partial(_se_gate_kernel, inv_hw=1.0 / float(HW)),
        out_shape=jax.ShapeDtypeStruct((N, C, HW), x.dtype),
        grid=(N,),
        in_specs=[
            pl.BlockSpec((1, C, HW), lambda i: (i, 0, 0)),
            pl.BlockSpec((Cr, C), lambda i: (0, 0)),
            pl.BlockSpec((Cr, 1), lambda i: (0, 0)),
            pl.BlockSpec((C, Cr), lambda i: (0, 0)),
            pl.BlockSpec((C, 1), lambda i: (0, 0)),
        ],
        out_specs=pl.BlockSpec((1, C, HW), lambda i: (i, 0, 0)),
        compiler_params=pltpu.CompilerParams(
            dimension_semantics=("parallel",),
            vmem_limit_bytes=64 << 20,
        ),
    )(x_flat, w1t, b1c, w2t, b2c)

    return out_flat.reshape(N, C, H, W)


# speedup vs baseline: 1.0699x; 1.0699x over previous
import functools

import jax
import jax.numpy as jnp
from jax.experimental import pallas as pl
from jax.experimental.pallas import tpu as pltpu

_NB = 4


def _se_gate_kernel(x_ref, w1_ref, b1_ref, w2_ref, b2_ref, o_ref, *, inv_hw):
    x = x_ref[...]

    pooled = jnp.sum(x, axis=2) * inv_hw

    h = jnp.dot(pooled, w1_ref[...],
                preferred_element_type=jnp.float32) + b1_ref[...]
    h = jnp.maximum(h, 0.0)
    y = jax.nn.sigmoid(
        jnp.dot(h, w2_ref[...],
                preferred_element_type=jnp.float32) + b2_ref[...])

    o_ref[...] = x * y[:, :, None]


def kernel(x, w1, b1, w2, b2):
    N, C, H, W = x.shape
    Cr = w1.shape[1]
    HW = H * W

    x_flat = x.reshape(N, C, HW)
    b1r = b1.reshape(1, Cr)
    b2r = b2.reshape(1, C)

    nb = _NB
    out_flat = pl.pallas_call(
        functools.partial(_se_gate_kernel, inv_hw=1.0 / float(HW)),
        out_shape=jax.ShapeDtypeStruct((N, C, HW), x.dtype),
        grid=(N // nb,),
        in_specs=[
            pl.BlockSpec((nb, C, HW), lambda i: (i, 0, 0)),
            pl.BlockSpec((C, Cr), lambda i: (0, 0)),
            pl.BlockSpec((1, Cr), lambda i: (0, 0)),
            pl.BlockSpec((Cr, C), lambda i: (0, 0)),
            pl.BlockSpec((1, C), lambda i: (0, 0)),
        ],
        out_specs=pl.BlockSpec((nb, C, HW), lambda i: (i, 0, 0)),
        compiler_params=pltpu.CompilerParams(
            dimension_semantics=("parallel",),
            vmem_limit_bytes=64 << 20,
        ),
    )(x_flat, w1, b1r, w2, b2r)

    return out_flat.reshape(N, C, H, W)
